# baseline (speedup 1.0000x reference)
import functools

import jax
import jax.numpy as jnp
from jax import lax
from jax.experimental import pallas as pl
from jax.experimental.pallas import tpu as pltpu

_VMEM_LIMIT = 48 * 1024 * 1024


def _silu(v):
    return v * (1.0 / (1.0 + jnp.exp(-v)))


def _group_allreduce(t, cpg):
    sh = 1
    while sh < cpg:
        t = t + jnp.roll(t, sh, axis=1)
        sh *= 2
    lane = lax.broadcasted_iota(jnp.int32, t.shape, 1)
    b = jnp.where(lane % cpg == cpg - 1, t, 0.0)
    sh = 1
    while sh < cpg:
        b = b + jnp.roll(b, -sh, axis=1)
        sh *= 2
    return b


def _gn_scale_shift(xf, gamma, beta, *, groups, eps):
    HW, C = xf.shape
    cpg = C // groups
    s = jnp.sum(xf, axis=0, keepdims=True)
    q = jnp.sum(xf * xf, axis=0, keepdims=True)
    sq = jnp.concatenate([s, q], axis=0)
    r = lax.broadcasted_iota(jnp.int32, (C, C), 0) // cpg
    cc = lax.broadcasted_iota(jnp.int32, (C, C), 1) // cpg
    mask = (r == cc).astype(jnp.float32)
    gsq = jnp.dot(sq, mask, preferred_element_type=jnp.float32)
    inv_n = 1.0 / float(HW * cpg)
    mean = gsq[0:1] * inv_n
    var = gsq[1:2] * inv_n - mean * mean
    scale = lax.rsqrt(var + eps) * gamma
    shift = beta - mean * scale
    return scale, shift


def _stage_shifts(v_bf, xsh, *, H, W, C):
    HW = H * W
    zrow = jnp.zeros((W, 3 * C), jnp.bfloat16)
    xsh[0:W, :] = zrow
    xsh[(H + 1) * W:(H + 2) * W, :] = zrow
    xsh[W:W + HW, C:2 * C] = v_bf
    col = lax.broadcasted_iota(jnp.int32, (HW, 1), 0) % W
    left = xsh[W - 1:W - 1 + HW, C:2 * C]
    xsh[W:W + HW, 0:C] = jnp.where(col == 0, jnp.bfloat16(0), left)
    right = xsh[W + 1:W + 1 + HW, C:2 * C]
    xsh[W:W + HW, 2 * C:3 * C] = jnp.where(col == W - 1, jnp.bfloat16(0), right)


def _conv3x3(xsh, w_ref, *, H, W):
    HW = H * W
    acc = None
    for dy in range(3):
        slab = xsh[dy * W:dy * W + HW, :]
        d = jnp.dot(slab, w_ref[dy], preferred_element_type=jnp.float32)
        acc = d if acc is None else acc + d
    return acc


def _stage4(v_bf, xsh, *, H, W, C):
    HW = H * W
    zrow = jnp.zeros((W, 4 * C), jnp.bfloat16)
    xsh[0:W, :] = zrow
    xsh[(H + 1) * W:(H + 2) * W, :] = zrow
    xsh[W:W + HW, C:2 * C] = v_bf
    col = lax.broadcasted_iota(jnp.int32, (HW, 1), 0) % W
    left = xsh[W - 1:W - 1 + HW, C:2 * C]
    left_val = jnp.where(col == 0, jnp.bfloat16(0), left)
    xsh[W:W + HW, 0:C] = left_val
    right = xsh[W + 1:W + 1 + HW, C:2 * C]
    xsh[W:W + HW, 2 * C:3 * C] = jnp.where(col == W - 1, jnp.bfloat16(0), right)
    xsh[0:HW, 3 * C:4 * C] = left_val
    xsh[HW:HW + W, 3 * C:4 * C] = jnp.zeros((W, C), jnp.bfloat16)


def _k1(x_ref, emb_ref, g_ref, be_ref, w_ref, cb_ref, wl_ref, bl_ref,
        h_ref, xsh, *, H, W, groups, eps, upb):
    e_all = _silu(emb_ref[:, 0, :]).astype(jnp.bfloat16)
    eo_all = jnp.dot(e_all, wl_ref[...],
                     preferred_element_type=jnp.float32)
    for j in range(upb):
        xt = x_ref[j]
        C = xt.shape[0]
        HW = H * W
        cpg = C // groups
        s = jnp.sum(xt, axis=1, keepdims=True)
        q = jnp.sum(xt * xt, axis=1, keepdims=True)
        sq = jnp.concatenate([s, q], axis=1)
        r = lax.broadcasted_iota(jnp.int32, (C, C), 0) // cpg
        cc = lax.broadcasted_iota(jnp.int32, (C, C), 1) // cpg
        mask = (r == cc).astype(jnp.float32)
        gsq = jnp.dot(mask, sq, preferred_element_type=jnp.float32)
        inv_n = 1.0 / float(HW * cpg)
        mean = gsq[:, 0:1] * inv_n
        var = gsq[:, 1:2] * inv_n - mean * mean
        scale = lax.rsqrt(var + eps) * g_ref[...]
        shift = be_ref[...] - mean * scale
        v = _silu(xt * scale + shift).astype(jnp.bfloat16)
        vt = jnp.transpose(v, (1, 0))
        _stage4(vt, xsh.at[j], H=H, W=W, C=C)
        s = xsh.at[j]
        acc = (jnp.dot(s[0:HW, 0:4 * C], w_ref[0:4 * C],
                       preferred_element_type=jnp.float32)
               + jnp.dot(s[W:W + HW, C:4 * C], w_ref[4 * C:7 * C],
                         preferred_element_type=jnp.float32)
               + jnp.dot(s[2 * W:2 * W + HW, C:3 * C], w_ref[7 * C:9 * C],
                         preferred_element_type=jnp.float32))
        eo = eo_all[j:j + 1]
        h_ref[j] = (acc + (cb_ref[...] + bl_ref[...] + eo)).astype(jnp.bfloat16)


def _k2(h_ref, g_ref, be_ref, w_ref, cb_ref, x_ref, wsc_ref, bsc_ref,
        o_ref, xsh, *, H, W, groups, eps, upb):
    for j in range(upb):
        hf = h_ref[j].astype(jnp.float32)
        scale, shift = _gn_scale_shift(hf, g_ref[...], be_ref[...],
                                       groups=groups, eps=eps)
        v = _silu(hf * scale + shift).astype(jnp.bfloat16)
        _stage_shifts(v, xsh.at[j], H=H, W=W, C=hf.shape[1])
        acc = _conv3x3(xsh.at[j], w_ref, H=H, W=W)
        sc = jnp.dot(x_ref[j], wsc_ref[...], preferred_element_type=jnp.float32)
        o_ref[j] = acc + sc + (cb_ref[...] + bsc_ref[...])


def kernel(x, emb, gn1_w, gn1_b, conv1_w, conv1_b, lin_w, lin_b,
           gn2_w, gn2_b, conv2_w, conv2_b, sc_w, sc_b):
    B, Cin, H, W = x.shape
    Cout = conv1_w.shape[0]
    E = emb.shape[-1]
    HW = H * W
    groups, eps = 32, 1e-5
    bf = jnp.bfloat16
    upb = 4 if B % 4 == 0 else (2 if B % 2 == 0 else 1)
    nsteps = B // upb

    x3 = x.reshape(B, Cin, HW)
    x_bf = jnp.transpose(x, (0, 2, 3, 1)).reshape(B, HW, Cin).astype(bf)
    w1r = jnp.transpose(conv1_w, (2, 3, 1, 0)).reshape(9 * Cin, Cout).astype(bf)
    w2r = jnp.transpose(conv2_w, (2, 3, 1, 0)).reshape(3, 3 * Cout, Cout).astype(bf)
    wl = jnp.transpose(lin_w, (1, 0)).astype(bf)
    wsc = sc_w[:, :, 0, 0].transpose(1, 0).astype(bf)
    b1r = conv1_b.reshape(1, Cout)
    blr = lin_b.reshape(1, Cout)
    b2r = conv2_b.reshape(1, Cout)
    bscr = sc_b.reshape(1, Cout)
    g1 = gn1_w.reshape(Cin, 1)
    be1 = gn1_b.reshape(Cin, 1)
    g2 = gn2_w.reshape(1, Cout)
    be2 = gn2_b.reshape(1, Cout)
    embr = emb.reshape(B, 1, E)

    k1 = functools.partial(_k1, H=H, W=W, groups=groups, eps=eps, upb=upb)
    h = pl.pallas_call(
        k1,
        out_shape=jax.ShapeDtypeStruct((B, HW, Cout), bf),
        grid_spec=pltpu.PrefetchScalarGridSpec(
            num_scalar_prefetch=0,
            grid=(nsteps,),
            in_specs=[
                pl.BlockSpec((upb, Cin, HW), lambda b: (b, 0, 0)),
                pl.BlockSpec((upb, 1, E), lambda b: (b, 0, 0)),
                pl.BlockSpec((Cin, 1), lambda b: (0, 0)),
                pl.BlockSpec((Cin, 1), lambda b: (0, 0)),
                pl.BlockSpec((9 * Cin, Cout), lambda b: (0, 0)),
                pl.BlockSpec((1, Cout), lambda b: (0, 0)),
                pl.BlockSpec((E, Cout), lambda b: (0, 0)),
                pl.BlockSpec((1, Cout), lambda b: (0, 0)),
            ],
            out_specs=pl.BlockSpec((upb, HW, Cout), lambda b: (b, 0, 0)),
            scratch_shapes=[pltpu.VMEM((upb, (H + 2) * W, 4 * Cin), bf)],
        ),
        compiler_params=pltpu.CompilerParams(
            dimension_semantics=("arbitrary",),
            vmem_limit_bytes=_VMEM_LIMIT),
    )(x3, embr, g1, be1, w1r, b1r, wl, blr)

    k2 = functools.partial(_k2, H=H, W=W, groups=groups, eps=eps, upb=upb)
    out = pl.pallas_call(
        k2,
        out_shape=jax.ShapeDtypeStruct((B, HW, Cout), jnp.float32),
        grid_spec=pltpu.PrefetchScalarGridSpec(
            num_scalar_prefetch=0,
            grid=(nsteps,),
            in_specs=[
                pl.BlockSpec((upb, HW, Cout), lambda b: (b, 0, 0)),
                pl.BlockSpec((1, Cout), lambda b: (0, 0)),
                pl.BlockSpec((1, Cout), lambda b: (0, 0)),
                pl.BlockSpec((3, 3 * Cout, Cout), lambda b: (0, 0, 0)),
                pl.BlockSpec((1, Cout), lambda b: (0, 0)),
                pl.BlockSpec((upb, HW, Cin), lambda b: (b, 0, 0)),
                pl.BlockSpec((Cin, Cout), lambda b: (0, 0)),
                pl.BlockSpec((1, Cout), lambda b: (0, 0)),
            ],
            out_specs=pl.BlockSpec((upb, HW, Cout), lambda b: (b, 0, 0)),
            scratch_shapes=[pltpu.VMEM((upb, (H + 2) * W, 3 * Cout), bf)],
        ),
        compiler_params=pltpu.CompilerParams(
            dimension_semantics=("arbitrary",),
            allow_input_fusion=[False] * 5 + [True, False, False],
            vmem_limit_bytes=_VMEM_LIMIT),
    )(h, g2, be2, w2r, b2r, x_bf, wsc, bscr)

    return jnp.transpose(out.reshape(B, H, W, Cout), (0, 3, 1, 2))


# speedup vs baseline: 1.2511x; 1.2511x over previous
import functools

import jax
import jax.numpy as jnp
from jax import lax
from jax.experimental import pallas as pl
from jax.experimental.pallas import tpu as pltpu

_VMEM_LIMIT = 48 * 1024 * 1024


def _silu(v):
    return v * (1.0 / (1.0 + jnp.exp(-v)))


def _gn_scale_shift(xf, gamma, beta, *, groups, eps):
    HW, C = xf.shape
    cpg = C // groups
    s = jnp.sum(xf, axis=0, keepdims=True)
    q = jnp.sum(xf * xf, axis=0, keepdims=True)
    sq = jnp.concatenate([s, q], axis=0)
    r = lax.broadcasted_iota(jnp.int32, (C, C), 0) // cpg
    cc = lax.broadcasted_iota(jnp.int32, (C, C), 1) // cpg
    mask = (r == cc).astype(jnp.float32)
    gsq = jnp.dot(sq, mask, preferred_element_type=jnp.float32)
    inv_n = 1.0 / float(HW * cpg)
    mean = gsq[0:1] * inv_n
    var = gsq[1:2] * inv_n - mean * mean
    scale = lax.rsqrt(var + eps) * gamma
    shift = beta - mean * scale
    return scale, shift


def _stage_shifts(v_bf, xsh, *, H, W, C):
    HW = H * W
    zrow = jnp.zeros((W, 3 * C), jnp.bfloat16)
    xsh[0:W, :] = zrow
    xsh[(H + 1) * W:(H + 2) * W, :] = zrow
    xsh[W:W + HW, C:2 * C] = v_bf
    col = lax.broadcasted_iota(jnp.int32, (HW, 1), 0) % W
    left = xsh[W - 1:W - 1 + HW, C:2 * C]
    xsh[W:W + HW, 0:C] = jnp.where(col == 0, jnp.bfloat16(0), left)
    right = xsh[W + 1:W + 1 + HW, C:2 * C]
    xsh[W:W + HW, 2 * C:3 * C] = jnp.where(col == W - 1, jnp.bfloat16(0), right)


def _conv3x3(xsh, w_ref, *, H, W):
    HW = H * W
    acc = None
    for dy in range(3):
        slab = xsh[dy * W:dy * W + HW, :]
        d = jnp.dot(slab, w_ref[dy], preferred_element_type=jnp.float32)
        acc = d if acc is None else acc + d
    return acc


def _stage4(v_bf, xsh, *, H, W, C):
    HW = H * W
    zrow = jnp.zeros((W, 4 * C), jnp.bfloat16)
    xsh[0:W, :] = zrow
    xsh[(H + 1) * W:(H + 2) * W, :] = zrow
    xsh[W:W + HW, C:2 * C] = v_bf
    col = lax.broadcasted_iota(jnp.int32, (HW, 1), 0) % W
    left = xsh[W - 1:W - 1 + HW, C:2 * C]
    left_val = jnp.where(col == 0, jnp.bfloat16(0), left)
    xsh[W:W + HW, 0:C] = left_val
    right = xsh[W + 1:W + 1 + HW, C:2 * C]
    xsh[W:W + HW, 2 * C:3 * C] = jnp.where(col == W - 1, jnp.bfloat16(0), right)
    xsh[0:HW, 3 * C:4 * C] = left_val
    xsh[HW:HW + W, 3 * C:4 * C] = jnp.zeros((W, C), jnp.bfloat16)


def _k1(x_ref, emb_ref, g_ref, be_ref, w_ref, cb_ref, wl_ref, bl_ref,
        h_ref, xsh, *, H, W, groups, eps, upb):
    e_all = _silu(emb_ref[:, 0, :]).astype(jnp.bfloat16)
    eo_all = jnp.dot(e_all, wl_ref[...],
                     preferred_element_type=jnp.float32)
    for j in range(upb):
        xf = x_ref[j]
        C = xf.shape[1]
        HW = H * W
        scale, shift = _gn_scale_shift(xf, g_ref[...], be_ref[...],
                                       groups=groups, eps=eps)
        v = _silu(xf * scale + shift).astype(jnp.bfloat16)
        _stage4(v, xsh.at[j], H=H, W=W, C=C)
        s = xsh.at[j]
        acc = (jnp.dot(s[0:HW, 0:4 * C], w_ref[0:4 * C],
                       preferred_element_type=jnp.float32)
               + jnp.dot(s[W:W + HW, C:4 * C], w_ref[4 * C:7 * C],
                         preferred_element_type=jnp.float32)
               + jnp.dot(s[2 * W:2 * W + HW, C:3 * C], w_ref[7 * C:9 * C],
                         preferred_element_type=jnp.float32))
        eo = eo_all[j:j + 1]
        h_ref[j] = (acc + (cb_ref[...] + bl_ref[...] + eo)).astype(jnp.bfloat16)


def _k2(h_ref, g_ref, be_ref, w_ref, cb_ref, x_ref, wsc_ref, bsc_ref,
        o_ref, xsh, *, H, W, groups, eps, upb):
    for j in range(upb):
        hf = h_ref[j].astype(jnp.float32)
        scale, shift = _gn_scale_shift(hf, g_ref[...], be_ref[...],
                                       groups=groups, eps=eps)
        v = _silu(hf * scale + shift).astype(jnp.bfloat16)
        _stage_shifts(v, xsh.at[j], H=H, W=W, C=hf.shape[1])
        acc = _conv3x3(xsh.at[j], w_ref, H=H, W=W)
        sc = jnp.dot(x_ref[j], wsc_ref[...], preferred_element_type=jnp.float32)
        o_ref[j] = acc + sc + (cb_ref[...] + bsc_ref[...])


def kernel(x, emb, gn1_w, gn1_b, conv1_w, conv1_b, lin_w, lin_b,
           gn2_w, gn2_b, conv2_w, conv2_b, sc_w, sc_b):
    B, Cin, H, W = x.shape
    Cout = conv1_w.shape[0]
    E = emb.shape[-1]
    HW = H * W
    groups, eps = 32, 1e-5
    bf = jnp.bfloat16
    upb = 4 if B % 4 == 0 else (2 if B % 2 == 0 else 1)
    nsteps = B // upb

    x_nhwc = jnp.transpose(x, (0, 2, 3, 1)).reshape(B, HW, Cin)
    x_bf = x_nhwc.astype(bf)
    w1r = jnp.transpose(conv1_w, (2, 3, 1, 0)).reshape(9 * Cin, Cout).astype(bf)
    w2r = jnp.transpose(conv2_w, (2, 3, 1, 0)).reshape(3, 3 * Cout, Cout).astype(bf)
    wl = jnp.transpose(lin_w, (1, 0)).astype(bf)
    wsc = sc_w[:, :, 0, 0].transpose(1, 0).astype(bf)
    b1r = conv1_b.reshape(1, Cout)
    blr = lin_b.reshape(1, Cout)
    b2r = conv2_b.reshape(1, Cout)
    bscr = sc_b.reshape(1, Cout)
    g1 = gn1_w.reshape(1, Cin)
    be1 = gn1_b.reshape(1, Cin)
    g2 = gn2_w.reshape(1, Cout)
    be2 = gn2_b.reshape(1, Cout)
    embr = emb.reshape(B, 1, E)

    k1 = functools.partial(_k1, H=H, W=W, groups=groups, eps=eps, upb=upb)
    h = pl.pallas_call(
        k1,
        out_shape=jax.ShapeDtypeStruct((B, HW, Cout), bf),
        grid_spec=pltpu.PrefetchScalarGridSpec(
            num_scalar_prefetch=0,
            grid=(nsteps,),
            in_specs=[
                pl.BlockSpec((upb, HW, Cin), lambda b: (b, 0, 0)),
                pl.BlockSpec((upb, 1, E), lambda b: (b, 0, 0)),
                pl.BlockSpec((1, Cin), lambda b: (0, 0)),
                pl.BlockSpec((1, Cin), lambda b: (0, 0)),
                pl.BlockSpec((9 * Cin, Cout), lambda b: (0, 0)),
                pl.BlockSpec((1, Cout), lambda b: (0, 0)),
                pl.BlockSpec((E, Cout), lambda b: (0, 0)),
                pl.BlockSpec((1, Cout), lambda b: (0, 0)),
            ],
            out_specs=pl.BlockSpec((upb, HW, Cout), lambda b: (b, 0, 0)),
            scratch_shapes=[pltpu.VMEM((upb, (H + 2) * W, 4 * Cin), bf)],
        ),
        compiler_params=pltpu.CompilerParams(
            dimension_semantics=("arbitrary",),
            vmem_limit_bytes=_VMEM_LIMIT),
    )(x_nhwc, embr, g1, be1, w1r, b1r, wl, blr)

    k2 = functools.partial(_k2, H=H, W=W, groups=groups, eps=eps, upb=upb)
    out = pl.pallas_call(
        k2,
        out_shape=jax.ShapeDtypeStruct((B, HW, Cout), jnp.float32),
        grid_spec=pltpu.PrefetchScalarGridSpec(
            num_scalar_prefetch=0,
            grid=(nsteps,),
            in_specs=[
                pl.BlockSpec((upb, HW, Cout), lambda b: (b, 0, 0)),
                pl.BlockSpec((1, Cout), lambda b: (0, 0)),
                pl.BlockSpec((1, Cout), lambda b: (0, 0)),
                pl.BlockSpec((3, 3 * Cout, Cout), lambda b: (0, 0, 0)),
                pl.BlockSpec((1, Cout), lambda b: (0, 0)),
                pl.BlockSpec((upb, HW, Cin), lambda b: (b, 0, 0)),
                pl.BlockSpec((Cin, Cout), lambda b: (0, 0)),
                pl.BlockSpec((1, Cout), lambda b: (0, 0)),
            ],
            out_specs=pl.BlockSpec((upb, HW, Cout), lambda b: (b, 0, 0)),
            scratch_shapes=[pltpu.VMEM((upb, (H + 2) * W, 3 * Cout), bf)],
        ),
        compiler_params=pltpu.CompilerParams(
            dimension_semantics=("arbitrary",),
            allow_input_fusion=[False] * 5 + [True, False, False],
            vmem_limit_bytes=_VMEM_LIMIT),
    )(h, g2, be2, w2r, b2r, x_bf, wsc, bscr)

    return jnp.transpose(out.reshape(B, H, W, Cout), (0, 3, 1, 2))
